# baseline (speedup 1.0000x reference)
import functools

import jax
import jax.numpy as jnp
from jax import lax
from jax.experimental import pallas as pl
from jax.experimental.pallas import tpu as pltpu
from jax.experimental.pallas import tpu_sc as plsc

N_CHANNELS = 192
N_KNOTS = 65
INV_GRID = 32.0

N = 4
H = W = 224
SLAB = H * W

NUM_CORES = 2
NUM_SUBCORES = 16
NW = NUM_CORES * NUM_SUBCORES

ROWS = 56
CHUNKS_PER_SLAB = H // ROWS
SLABS = N * N_CHANNELS
SLABS_PER_W = SLABS // NW
CHUNKS_PER_W = SLABS_PER_W * CHUNKS_PER_SLAB
VPR = W // 16

TAB = N_CHANNELS * N_KNOTS


def _body(x_hbm, btab_hbm, d0tab_hbm, d2tab_hbm, out_hbm,
          btab, d0tab, d2tab, inb0, inb1, outb0, outb1,
          si0, si1, so0, so1):
    cid = lax.axis_index("c")
    sid = lax.axis_index("s")
    wid = sid * NUM_CORES + cid

    pltpu.sync_copy(btab_hbm, btab)
    pltpu.sync_copy(d0tab_hbm, d0tab)
    pltpu.sync_copy(d2tab_hbm, d2tab)

    base_chunk = wid * CHUNKS_PER_W

    def src_of(g):
        slab = (base_chunk + g) // CHUNKS_PER_SLAB
        r0 = (base_chunk + g) % CHUNKS_PER_SLAB * ROWS
        n = slab // N_CHANNELS
        c = slab % N_CHANNELS
        return n, c, r0

    inbufs = (inb0, inb1)
    outbufs = (outb0, outb1)
    sis = (si0, si1)
    sos = (so0, so1)

    def start_in(g, b):
        n, c, r0 = src_of(g)
        pltpu.async_copy(
            x_hbm.at[n, c, pl.ds(r0, ROWS), :], inbufs[b], sis[b])

    start_in(0, 0)
    start_in(1, 1)

    def chunk_pair(j, carry):
        for b in range(2):
            g = j * 2 + b
            inbuf, outbuf, si, so = inbufs[b], outbufs[b], sis[b], sos[b]
            n, c, r0 = src_of(g)
            c65 = c * N_KNOTS

            pltpu.make_async_copy(
                x_hbm.at[n, c, pl.ds(r0, ROWS), :], inbuf, si).wait()

            @pl.when(j >= 1)
            def _():
                pltpu.make_async_copy(
                    outbuf, out_hbm.at[n, c, pl.ds(r0, ROWS), :], so).wait()

            @plsc.parallel_loop(0, ROWS, unroll=2)
            def _(r):
                for v in range(VPR):
                    col = v * 16
                    x = inbuf[r, pl.ds(col, 16)]
                    xb = x * INV_GRID + (N_KNOTS // 2) * 1.0
                    xcb = jnp.minimum(jnp.maximum(xb, 0.0),
                                      (N_KNOTS - 3) * 1.0)
                    ti = xcb.astype(jnp.int32)
                    s = xb - ti.astype(jnp.float32)
                    ki = ti + c65
                    u = s - 1.0
                    gb = plsc.load_gather(btab, [ki])
                    g0 = plsc.load_gather(d0tab, [ki])
                    g2 = plsc.load_gather(d2tab, [ki])
                    outbuf[r, pl.ds(col, 16)] = gb + (u * u) * g0 + (s * s) * g2

            pltpu.async_copy(outbuf, out_hbm.at[n, c, pl.ds(r0, ROWS), :], so)

            @pl.when(j < CHUNKS_PER_W // 2 - 1)
            def _():
                n2, c2, r2 = src_of(g + 2)
                pltpu.async_copy(
                    x_hbm.at[n2, c2, pl.ds(r2, ROWS), :], inbuf, si)
        return carry

    lax.fori_loop(0, CHUNKS_PER_W // 2, chunk_pair, 0)

    n, c, r0 = src_of(CHUNKS_PER_W - 2)
    pltpu.make_async_copy(
        outb0, out_hbm.at[n, c, pl.ds(r0, ROWS), :], so0).wait()
    n, c, r0 = src_of(CHUNKS_PER_W - 1)
    pltpu.make_async_copy(
        outb1, out_hbm.at[n, c, pl.ds(r0, ROWS), :], so1).wait()


_mesh = plsc.VectorSubcoreMesh(core_axis_name="c", subcore_axis_name="s")

_sc_call = functools.partial(
    pl.kernel,
    out_type=jax.ShapeDtypeStruct((N, N_CHANNELS, H, W), jnp.float32),
    mesh=_mesh,
    compiler_params=pltpu.CompilerParams(
        needs_layout_passes=False, use_tc_tiling_on_sc=True),
    scratch_types=[
        pltpu.VMEM((TAB,), jnp.float32),
        pltpu.VMEM((TAB,), jnp.float32),
        pltpu.VMEM((TAB,), jnp.float32),
        pltpu.VMEM((ROWS, W), jnp.float32),
        pltpu.VMEM((ROWS, W), jnp.float32),
        pltpu.VMEM((ROWS, W), jnp.float32),
        pltpu.VMEM((ROWS, W), jnp.float32),
        pltpu.SemaphoreType.DMA,
        pltpu.SemaphoreType.DMA,
        pltpu.SemaphoreType.DMA,
        pltpu.SemaphoreType.DMA,
    ],
)(_body)


@jax.jit
def kernel(x, coefficients_vect, zero_knot_indexes):
    base = zero_knot_indexes.astype(jnp.int32) - (N_KNOTS // 2)
    offs = base[:, None] + jnp.arange(N_KNOTS, dtype=jnp.int32)[None, :]
    c0 = jnp.take(coefficients_vect, offs)
    c1 = jnp.take(coefficients_vect, offs + 1)
    c2 = jnp.take(coefficients_vect, offs + 2)
    btab = c1.reshape(-1)
    d0tab = (0.5 * (c0 - c1)).reshape(-1)
    d2tab = (0.5 * (c2 - c1)).reshape(-1)
    return _sc_call(x, btab, d0tab, d2tab)


# speedup vs baseline: 1.2096x; 1.2096x over previous
import functools

import jax
import jax.numpy as jnp
from jax import lax
from jax.experimental import pallas as pl
from jax.experimental.pallas import tpu as pltpu
from jax.experimental.pallas import tpu_sc as plsc

N_CHANNELS = 192
N_KNOTS = 65
INV_GRID = 32.0

N = 4
H = W = 224
SLAB = H * W

NUM_CORES = 2
NUM_SUBCORES = 16
NW = NUM_CORES * NUM_SUBCORES

ROWS = 56
CHUNKS_PER_SLAB = H // ROWS
SLABS = N * N_CHANNELS
SLABS_PER_W = SLABS // NW
CHUNKS_PER_W = SLABS_PER_W * CHUNKS_PER_SLAB
VPR = W // 16

TAB = N_CHANNELS * N_KNOTS


def _body(x_hbm, btab_hbm, d0tab_hbm, d2tab_hbm, out_hbm,
          btab, d0tab, d2tab, inb0, inb1, outb0, outb1,
          si0, si1, so0, so1):
    cid = lax.axis_index("c")
    sid = lax.axis_index("s")
    wid = sid * NUM_CORES + cid

    pltpu.sync_copy(btab_hbm, btab)
    pltpu.sync_copy(d0tab_hbm, d0tab)
    pltpu.sync_copy(d2tab_hbm, d2tab)

    base_chunk = wid * CHUNKS_PER_W

    def src_of(g):
        slab = (base_chunk + g) // CHUNKS_PER_SLAB
        r0 = (base_chunk + g) % CHUNKS_PER_SLAB * ROWS
        n = slab // N_CHANNELS
        c = slab % N_CHANNELS
        return n, c, r0

    inbufs = (inb0, inb1)
    outbufs = (outb0, outb1)
    sis = (si0, si1)
    sos = (so0, so1)

    def start_in(g, b):
        n, c, r0 = src_of(g)
        pltpu.async_copy(
            x_hbm.at[n, c, pl.ds(r0, ROWS), :], inbufs[b], sis[b])

    start_in(0, 0)
    start_in(1, 1)

    def chunk_pair(j, carry):
        for b in range(2):
            g = j * 2 + b
            inbuf, outbuf, si, so = inbufs[b], outbufs[b], sis[b], sos[b]
            n, c, r0 = src_of(g)
            c65 = c * N_KNOTS

            pltpu.make_async_copy(
                x_hbm.at[n, c, pl.ds(r0, ROWS), :], inbuf, si).wait()

            @pl.when(j >= 1)
            def _():
                pltpu.make_async_copy(
                    outbuf, out_hbm.at[n, c, pl.ds(r0, ROWS), :], so).wait()

            @plsc.parallel_loop(0, ROWS, unroll=1)
            def _(r):
                for v in range(VPR):
                    col = v * 16
                    x = inbuf[r, pl.ds(col, 16)]
                    xb = x * INV_GRID + (N_KNOTS // 2) * 1.0
                    xcb = jnp.minimum(jnp.maximum(xb, 0.0),
                                      (N_KNOTS - 3) * 1.0)
                    ti = xcb.astype(jnp.int32)
                    s = xb - ti.astype(jnp.float32)
                    ki = ti + c65
                    g0 = plsc.load_gather(btab, [ki])
                    g1 = plsc.load_gather(d0tab, [ki])
                    g2 = plsc.load_gather(d2tab, [ki])
                    outbuf[r, pl.ds(col, 16)] = g0 + s * (g1 + s * g2)

            pltpu.async_copy(outbuf, out_hbm.at[n, c, pl.ds(r0, ROWS), :], so)

            @pl.when(j < CHUNKS_PER_W // 2 - 1)
            def _():
                n2, c2, r2 = src_of(g + 2)
                pltpu.async_copy(
                    x_hbm.at[n2, c2, pl.ds(r2, ROWS), :], inbuf, si)
        return carry

    lax.fori_loop(0, CHUNKS_PER_W // 2, chunk_pair, 0)

    n, c, r0 = src_of(CHUNKS_PER_W - 2)
    pltpu.make_async_copy(
        outb0, out_hbm.at[n, c, pl.ds(r0, ROWS), :], so0).wait()
    n, c, r0 = src_of(CHUNKS_PER_W - 1)
    pltpu.make_async_copy(
        outb1, out_hbm.at[n, c, pl.ds(r0, ROWS), :], so1).wait()


_mesh = plsc.VectorSubcoreMesh(core_axis_name="c", subcore_axis_name="s")

_sc_call = functools.partial(
    pl.kernel,
    out_type=jax.ShapeDtypeStruct((N, N_CHANNELS, H, W), jnp.float32),
    mesh=_mesh,
    compiler_params=pltpu.CompilerParams(
        needs_layout_passes=False, use_tc_tiling_on_sc=True),
    scratch_types=[
        pltpu.VMEM((TAB,), jnp.float32),
        pltpu.VMEM((TAB,), jnp.float32),
        pltpu.VMEM((TAB,), jnp.float32),
        pltpu.VMEM((ROWS, W), jnp.float32),
        pltpu.VMEM((ROWS, W), jnp.float32),
        pltpu.VMEM((ROWS, W), jnp.float32),
        pltpu.VMEM((ROWS, W), jnp.float32),
        pltpu.SemaphoreType.DMA,
        pltpu.SemaphoreType.DMA,
        pltpu.SemaphoreType.DMA,
        pltpu.SemaphoreType.DMA,
    ],
)(_body)


@jax.jit
def kernel(x, coefficients_vect, zero_knot_indexes):
    base = zero_knot_indexes.astype(jnp.int32) - (N_KNOTS // 2)
    offs = base[:, None] + jnp.arange(N_KNOTS, dtype=jnp.int32)[None, :]
    c0 = jnp.take(coefficients_vect, offs)
    c1 = jnp.take(coefficients_vect, offs + 1)
    c2 = jnp.take(coefficients_vect, offs + 2)
    btab = (0.5 * (c0 + c1)).reshape(-1)
    d0tab = (c1 - c0).reshape(-1)
    d2tab = (0.5 * (c0 + c2) - c1).reshape(-1)
    return _sc_call(x, btab, d0tab, d2tab)


# speedup vs baseline: 1.2121x; 1.0020x over previous
import functools

import jax
import jax.numpy as jnp
from jax import lax
from jax.experimental import pallas as pl
from jax.experimental.pallas import tpu as pltpu
from jax.experimental.pallas import tpu_sc as plsc

N_CHANNELS = 192
N_KNOTS = 65
INV_GRID = 32.0

N = 4
H = W = 224
SLAB = H * W

NUM_CORES = 2
NUM_SUBCORES = 16
NW = NUM_CORES * NUM_SUBCORES

ROWS = 112
CHUNKS_PER_SLAB = H // ROWS
SLABS = N * N_CHANNELS
SLABS_PER_W = SLABS // NW
CHUNKS_PER_W = SLABS_PER_W * CHUNKS_PER_SLAB
VPR = W // 16

TAB = N_CHANNELS * N_KNOTS


def _body(x_hbm, btab_hbm, d0tab_hbm, d2tab_hbm, out_hbm,
          btab, d0tab, d2tab, buf0, buf1, buf2,
          si0, si1, si2, so0, so1, so2):
    cid = lax.axis_index("c")
    sid = lax.axis_index("s")
    wid = sid * NUM_CORES + cid

    pltpu.sync_copy(btab_hbm, btab)
    pltpu.sync_copy(d0tab_hbm, d0tab)
    pltpu.sync_copy(d2tab_hbm, d2tab)

    base_chunk = wid * CHUNKS_PER_W

    def src_of(g):
        slab = (base_chunk + g) // CHUNKS_PER_SLAB
        r0 = (base_chunk + g) % CHUNKS_PER_SLAB * ROWS
        n = slab // N_CHANNELS
        c = slab % N_CHANNELS
        return n, c, r0

    bufs = (buf0, buf1, buf2)
    sis = (si0, si1, si2)
    sos = (so0, so1, so2)

    def start_in(g, b):
        n, c, r0 = src_of(g)
        pltpu.async_copy(
            x_hbm.at[n, c, pl.ds(r0, ROWS), :], bufs[b], sis[b])

    start_in(0, 0)
    start_in(1, 1)

    def chunk_triple(j, carry):
        for b in range(3):
            g = j * 3 + b
            buf, si, so = bufs[b], sis[b], sos[b]
            n, c, r0 = src_of(g)
            c65 = c * N_KNOTS

            pltpu.make_async_copy(
                x_hbm.at[n, c, pl.ds(r0, ROWS), :], buf, si).wait()

            @plsc.parallel_loop(0, ROWS, unroll=1)
            def _(r):
                for v in range(VPR):
                    col = v * 16
                    x = buf[r, pl.ds(col, 16)]
                    xb = x * INV_GRID + (N_KNOTS // 2) * 1.0
                    xcb = jnp.minimum(jnp.maximum(xb, 0.0),
                                      (N_KNOTS - 3) * 1.0)
                    ti = xcb.astype(jnp.int32)
                    s = xb - ti.astype(jnp.float32)
                    ki = ti + c65
                    g0 = plsc.load_gather(btab, [ki])
                    g1 = plsc.load_gather(d0tab, [ki])
                    g2 = plsc.load_gather(d2tab, [ki])
                    buf[r, pl.ds(col, 16)] = g0 + s * (g1 + s * g2)

            pltpu.async_copy(buf, out_hbm.at[n, c, pl.ds(r0, ROWS), :], so)

            pb = (b + 2) % 3
            if b == 0:
                @pl.when(j > 0)
                def _():
                    np_, cp, rp = src_of(g - 1)
                    pltpu.make_async_copy(
                        bufs[pb], out_hbm.at[np_, cp, pl.ds(rp, ROWS), :],
                        sos[pb]).wait()

                start_in(g + 2, pb)
            else:
                np_, cp, rp = src_of(g - 1)
                pltpu.make_async_copy(
                    bufs[pb], out_hbm.at[np_, cp, pl.ds(rp, ROWS), :],
                    sos[pb]).wait()

                @pl.when(j < CHUNKS_PER_W // 3 - 1)
                def _():
                    start_in(g + 2, pb)
        return carry

    lax.fori_loop(0, CHUNKS_PER_W // 3, chunk_triple, 0)

    n, c, r0 = src_of(CHUNKS_PER_W - 1)
    pltpu.make_async_copy(
        bufs[(CHUNKS_PER_W - 1) % 3],
        out_hbm.at[n, c, pl.ds(r0, ROWS), :],
        sos[(CHUNKS_PER_W - 1) % 3]).wait()


_mesh = plsc.VectorSubcoreMesh(core_axis_name="c", subcore_axis_name="s")

_sc_call = functools.partial(
    pl.kernel,
    out_type=jax.ShapeDtypeStruct((N, N_CHANNELS, H, W), jnp.float32),
    mesh=_mesh,
    compiler_params=pltpu.CompilerParams(
        needs_layout_passes=False, use_tc_tiling_on_sc=True),
    scratch_types=[
        pltpu.VMEM((TAB,), jnp.float32),
        pltpu.VMEM((TAB,), jnp.float32),
        pltpu.VMEM((TAB,), jnp.float32),
        pltpu.VMEM((ROWS, W), jnp.float32),
        pltpu.VMEM((ROWS, W), jnp.float32),
        pltpu.VMEM((ROWS, W), jnp.float32),
        pltpu.SemaphoreType.DMA,
        pltpu.SemaphoreType.DMA,
        pltpu.SemaphoreType.DMA,
        pltpu.SemaphoreType.DMA,
        pltpu.SemaphoreType.DMA,
        pltpu.SemaphoreType.DMA,
    ],
)(_body)


@jax.jit
def kernel(x, coefficients_vect, zero_knot_indexes):
    base = zero_knot_indexes.astype(jnp.int32) - (N_KNOTS // 2)
    offs = base[:, None] + jnp.arange(N_KNOTS, dtype=jnp.int32)[None, :]
    c0 = jnp.take(coefficients_vect, offs)
    c1 = jnp.take(coefficients_vect, offs + 1)
    c2 = jnp.take(coefficients_vect, offs + 2)
    btab = (0.5 * (c0 + c1)).reshape(-1)
    d0tab = (c1 - c0).reshape(-1)
    d2tab = (0.5 * (c0 + c2) - c1).reshape(-1)
    return _sc_call(x, btab, d0tab, d2tab)
